# baseline (speedup 1.0000x reference)
import functools

import jax
import jax.numpy as jnp
from jax import lax
from jax.experimental import pallas as pl
from jax.experimental.pallas import tpu as pltpu
from jax.experimental.pallas import tpu_sc as plsc

_EMBED = 128
_NC, _NS = 2, 16
_NW = _NC * _NS
_G = 128
_NB = 5
_D = 4


def _sc_gather(idx_t, table):
    hist, batch = idx_t.shape
    nb = batch * hist
    assert batch % (_NW * _G) == 0 and hist % _NB == 0
    mesh = plsc.VectorSubcoreMesh(
        core_axis_name="c", subcore_axis_name="s",
        num_cores=_NC, num_subcores=_NS)

    @functools.partial(
        pl.kernel,
        out_type=jax.ShapeDtypeStruct((hist, batch, _EMBED), jnp.float32),
        mesh=mesh,
        compiler_params=pltpu.CompilerParams(use_tc_tiling_on_sc=True),
        scratch_types=[
            pltpu.VMEM((hist, _G), jnp.int32),
            [pltpu.VMEM((_G, _EMBED), jnp.float32) for _ in range(_NB)],
            [pltpu.SemaphoreType.DMA for _ in range(_NB)],
            [pltpu.SemaphoreType.DMA for _ in range(_NB)],
        ],
    )
    def k(idx_hbm, table_hbm, out3_hbm, idx_v, bufs, gsems, ssems):
        wid = lax.axis_index("s") * _NC + lax.axis_index("c")
        nbase = wid * _G
        out_hbm = out3_hbm.reshape(nb, _EMBED)
        pltpu.sync_copy(idx_hbm.at[:, pl.ds(nbase, _G)], idx_v)

        for i in range(_D):
            pltpu.async_copy(table_hbm.at[idx_v.at[i]], bufs[i], gsems[i])

        @pl.loop(0, hist, step=_NB)
        def _outer(t0):
            for i in range(_NB):
                h = t0 + i
                pltpu.make_async_copy(
                    table_hbm.at[idx_v.at[0]], bufs[i], gsems[i]).wait()
                pltpu.async_copy(
                    bufs[i], out_hbm.at[pl.ds(h * batch + nbase, _G)],
                    ssems[i])
                hn = h + _D
                bn = (i + _D) % _NB

                @pl.when(hn < hist)
                def _():
                    @pl.when(hn >= _NB)
                    def _():
                        pltpu.make_async_copy(
                            bufs[bn], out_hbm.at[pl.ds(nbase, _G)],
                            ssems[bn]).wait()

                    pltpu.async_copy(
                        table_hbm.at[idx_v.at[hn]], bufs[bn], gsems[bn])

        for i in range(_NB):
            pltpu.make_async_copy(
                bufs[i], out_hbm.at[pl.ds(nbase, _G)], ssems[i]).wait()

    return k(idx_t, table)


def kernel(inputs, embedding_variable):
    idx_t = inputs.T.astype(jnp.int32)
    out_t = _sc_gather(idx_t, embedding_variable)
    return out_t.transpose(1, 0, 2)


# speedup vs baseline: 1.0019x; 1.0019x over previous
import functools

import jax
import jax.numpy as jnp
from jax import lax
from jax.experimental import pallas as pl
from jax.experimental.pallas import tpu as pltpu
from jax.experimental.pallas import tpu_sc as plsc

_EMBED = 128
_NC, _NS = 2, 16
_NW = _NC * _NS
_G = 128
_NB = 5
_D = 3


def _sc_gather(idx_t, table):
    hist, batch = idx_t.shape
    nb = batch * hist
    assert batch % (_NW * _G) == 0 and hist % _NB == 0
    mesh = plsc.VectorSubcoreMesh(
        core_axis_name="c", subcore_axis_name="s",
        num_cores=_NC, num_subcores=_NS)

    @functools.partial(
        pl.kernel,
        out_type=jax.ShapeDtypeStruct((hist, batch, _EMBED), jnp.float32),
        mesh=mesh,
        compiler_params=pltpu.CompilerParams(use_tc_tiling_on_sc=True),
        scratch_types=[
            pltpu.VMEM((hist, _G), jnp.int32),
            [pltpu.VMEM((_G, _EMBED), jnp.float32) for _ in range(_NB)],
            [pltpu.SemaphoreType.DMA for _ in range(_NB)],
            [pltpu.SemaphoreType.DMA for _ in range(_NB)],
        ],
    )
    def k(idx_hbm, table_hbm, out3_hbm, idx_v, bufs, gsems, ssems):
        wid = lax.axis_index("s") * _NC + lax.axis_index("c")
        nbase = wid * _G
        out_hbm = out3_hbm.reshape(nb, _EMBED)
        pltpu.sync_copy(idx_hbm.at[:, pl.ds(nbase, _G)], idx_v)

        for i in range(_D):
            pltpu.async_copy(table_hbm.at[idx_v.at[i]], bufs[i], gsems[i])

        @pl.loop(0, hist, step=_NB)
        def _outer(t0):
            for i in range(_NB):
                h = t0 + i
                pltpu.make_async_copy(
                    table_hbm.at[idx_v.at[0]], bufs[i], gsems[i]).wait()
                pltpu.async_copy(
                    bufs[i], out_hbm.at[pl.ds(h * batch + nbase, _G)],
                    ssems[i])
                hn = h + _D
                bn = (i + _D) % _NB

                @pl.when(hn < hist)
                def _():
                    @pl.when(hn >= _NB)
                    def _():
                        pltpu.make_async_copy(
                            bufs[bn], out_hbm.at[pl.ds(nbase, _G)],
                            ssems[bn]).wait()

                    pltpu.async_copy(
                        table_hbm.at[idx_v.at[hn]], bufs[bn], gsems[bn])

        for i in range(_NB):
            pltpu.make_async_copy(
                bufs[i], out_hbm.at[pl.ds(nbase, _G)], ssems[i]).wait()

    return k(idx_t, table)


def kernel(inputs, embedding_variable):
    idx_t = inputs.T.astype(jnp.int32)
    out_t = _sc_gather(idx_t, embedding_variable)
    return out_t.transpose(1, 0, 2)
